# Initial kernel scaffold; baseline (speedup 1.0000x reference)
import jax
import jax.numpy as jnp
from jax.experimental import pallas as pl


def kernel(x, e, W1, b1, W2, b2):
    raise NotImplementedError("write your pallas kernel here")


# speedup vs baseline: 13.0005x; 13.0005x over previous
import functools

import jax
import jax.numpy as jnp
from jax import lax
from jax.experimental import pallas as pl
from jax.experimental.pallas import tpu as pltpu
from jax.experimental.pallas import tpu_sc as plsc

N = 10000
E = 320000
D = 128

NC = 2
NS = 16
NW = NC * NS
N_PAD = 10240
EPT = E // NW
CH = 80
NCHUNK = EPT // CH
RPT = N_PAD // NS

BN = 1280
GRID = N_PAD // BN

_MESH = plsc.VectorSubcoreMesh(core_axis_name="c", subcore_axis_name="s")


@functools.partial(
    pl.kernel,
    out_type=jax.ShapeDtypeStruct((NC, N_PAD), jnp.float32),
    mesh=_MESH,
    scratch_types=[
        pltpu.VMEM((CH,), jnp.int32),
        pltpu.VMEM((CH,), jnp.float32),
        pltpu.VMEM_SHARED((N_PAD,), jnp.float32),
    ],
)
def _deg_kernel(dst_hbm, zeros_hbm, out_hbm, dst_v, ones_v, deg_sh):
    c = lax.axis_index("c")
    s = lax.axis_index("s")
    wid = c * NS + s
    for i in range(CH // 16):
        ones_v[pl.ds(i * 16, 16)] = jnp.ones((16,), jnp.float32)
    pltpu.sync_copy(zeros_hbm, deg_sh.at[pl.ds(s * RPT, RPT)])
    plsc.subcore_barrier()

    def body(i, carry):
        base = wid * EPT + i * CH
        pltpu.sync_copy(dst_hbm.at[pl.ds(base, CH)], dst_v)
        pltpu.sync_copy(ones_v, deg_sh.at[dst_v], add=True)
        return carry

    lax.fori_loop(0, NCHUNK, body, 0)
    plsc.subcore_barrier()
    pltpu.sync_copy(deg_sh.at[pl.ds(s * RPT, RPT)],
                    out_hbm.at[c, pl.ds(s * RPT, RPT)])


@functools.partial(
    pl.kernel,
    out_type=jax.ShapeDtypeStruct((NC, N_PAD, D), jnp.float32),
    mesh=_MESH,
    scratch_types=[
        pltpu.VMEM((CH,), jnp.int32),
        pltpu.VMEM((CH,), jnp.int32),
        pltpu.VMEM((CH, D), jnp.float32),
        pltpu.VMEM_SHARED((N_PAD, D), jnp.float32),
        pltpu.SemaphoreType.DMA,
    ],
)
def _scatter_kernel(g_hbm, src_hbm, dst_hbm, zeros_hbm, out_hbm,
                    src_v, dst_v, rows_v, acc_sh, sem):
    c = lax.axis_index("c")
    s = lax.axis_index("s")
    wid = c * NS + s
    pltpu.sync_copy(zeros_hbm, acc_sh.at[pl.ds(s * RPT, RPT)])
    plsc.subcore_barrier()

    def body(i, carry):
        base = wid * EPT + i * CH
        pltpu.sync_copy(src_hbm.at[pl.ds(base, CH)], src_v)
        pltpu.sync_copy(dst_hbm.at[pl.ds(base, CH)], dst_v)
        pltpu.async_copy(g_hbm.at[src_v], rows_v, sem).wait()
        pltpu.sync_copy(rows_v, acc_sh.at[dst_v], add=True)
        return carry

    lax.fori_loop(0, NCHUNK, body, 0)
    plsc.subcore_barrier()
    pltpu.sync_copy(acc_sh.at[pl.ds(s * RPT, RPT)],
                    out_hbm.at[c, pl.ds(s * RPT, RPT)])


def _k1_body(x_ref, w_ref, degp_ref, o_ref):
    deg = degp_ref[0] + degp_ref[1] + 1.0
    dis = lax.rsqrt(deg)
    h = jnp.dot(x_ref[...], w_ref[...], preferred_element_type=jnp.float32)
    o_ref[...] = h * dis


def _k3_body(acc_ref, g_ref, degp_ref, b_ref, w_ref, o_ref):
    deg = degp_ref[0] + degp_ref[1] + 1.0
    dis = lax.rsqrt(deg)
    t = dis * (acc_ref[0] + acc_ref[1] + g_ref[...]) + b_ref[...]
    t = jnp.maximum(t, 0.0)
    h = jnp.dot(t, w_ref[...], preferred_element_type=jnp.float32)
    o_ref[...] = h * dis


def _k5_body(acc_ref, g_ref, degp_ref, b_ref, o_ref):
    deg = degp_ref[0] + degp_ref[1] + 1.0
    dis = lax.rsqrt(deg)
    t = dis * (acc_ref[0] + acc_ref[1] + g_ref[...]) + b_ref[...]
    o_ref[...] = jnp.maximum(t, 0.0)


_ROWS = pl.BlockSpec((BN, D), lambda i: (i, 0))
_ACC = pl.BlockSpec((NC, BN, D), lambda i: (0, i, 0))
_DEGP = pl.BlockSpec((NC, BN, 1), lambda i: (0, i, 0))
_MAT = pl.BlockSpec((D, D), lambda i: (0, 0))
_BIAS = pl.BlockSpec((1, D), lambda i: (0, 0))

_k1 = pl.pallas_call(
    _k1_body, grid=(GRID,),
    in_specs=[_ROWS, _MAT, _DEGP],
    out_specs=_ROWS,
    out_shape=jax.ShapeDtypeStruct((N_PAD, D), jnp.float32),
)
_k3 = pl.pallas_call(
    _k3_body, grid=(GRID,),
    in_specs=[_ACC, _ROWS, _DEGP, _BIAS, _MAT],
    out_specs=_ROWS,
    out_shape=jax.ShapeDtypeStruct((N_PAD, D), jnp.float32),
)
_k5 = pl.pallas_call(
    _k5_body, grid=(GRID,),
    in_specs=[_ACC, _ROWS, _DEGP, _BIAS],
    out_specs=_ROWS,
    out_shape=jax.ShapeDtypeStruct((N_PAD, D), jnp.float32),
)


def kernel(x, e, W1, b1, W2, b2):
    src = e[0]
    dst = e[1]
    x_pad = jnp.pad(x, ((0, N_PAD - N), (0, 0)))
    zeros1 = jnp.zeros((RPT,), jnp.float32)
    zeros2 = jnp.zeros((RPT, D), jnp.float32)

    degp = _deg_kernel(dst, zeros1)
    degp = degp.reshape(NC, N_PAD, 1)
    b1r = b1.reshape(1, D)
    b2r = b2.reshape(1, D)

    g1 = _k1(x_pad, W1, degp)
    acc1 = _scatter_kernel(g1, src, dst, zeros2)
    g2 = _k3(acc1, g1, degp, b1r, W2)
    acc2 = _scatter_kernel(g2, src, dst, zeros2)
    out = _k5(acc2, g2, degp, b2r)
    return out[:N]


# speedup vs baseline: 26.2217x; 2.0170x over previous
import functools

import jax
import jax.numpy as jnp
from jax import lax
from jax.experimental import pallas as pl
from jax.experimental.pallas import tpu as pltpu
from jax.experimental.pallas import tpu_sc as plsc

N = 10000
E = 320000
D = 128
HD = D // 2

NC = 2
NS = 16
NW = NC * NS
N_PAD = 10240
RPT = N_PAD // NS

EPS = E // NS
CH = 125
NCHUNK = EPS // CH

EPT = E // NW
CHD = 125
NCHUNK_D = EPT // CHD

BN = 1280
GRID = N_PAD // BN

_MESH = plsc.VectorSubcoreMesh(core_axis_name="c", subcore_axis_name="s")


@functools.partial(
    pl.kernel,
    out_type=jax.ShapeDtypeStruct((NC, N_PAD), jnp.float32),
    mesh=_MESH,
    scratch_types=[
        pltpu.VMEM((NCHUNK_D, CHD), jnp.int32),
        pltpu.VMEM((CHD,), jnp.float32),
        pltpu.VMEM_SHARED((N_PAD,), jnp.float32),
        pltpu.SemaphoreType.DMA,
    ],
)
def _deg_kernel(dst_hbm, zeros_hbm, out_hbm, dst_all, ones_v, deg_sh, sem):
    c = lax.axis_index("c")
    s = lax.axis_index("s")
    wid = c * NS + s
    for i in range(0, CHD - 15, 16):
        ones_v[pl.ds(i, 16)] = jnp.ones((16,), jnp.float32)
    ones_v[pl.ds(CHD - 16, 16)] = jnp.ones((16,), jnp.float32)
    pltpu.sync_copy(dst_hbm.at[wid], dst_all)
    pltpu.sync_copy(zeros_hbm, deg_sh.at[pl.ds(s * RPT, RPT)])
    plsc.subcore_barrier()

    def fire(i, carry):
        pltpu.async_copy(ones_v, deg_sh.at[dst_all.at[i]], sem, add=True)
        return carry

    lax.fori_loop(0, NCHUNK_D, fire, 0)

    def drain(i, carry):
        pltpu.make_async_copy(ones_v, deg_sh.at[dst_all.at[0]], sem).wait()
        return carry

    lax.fori_loop(0, NCHUNK_D, drain, 0)
    plsc.subcore_barrier()
    pltpu.sync_copy(deg_sh.at[pl.ds(s * RPT, RPT)],
                    out_hbm.at[c, pl.ds(s * RPT, RPT)])


@functools.partial(
    pl.kernel,
    out_type=jax.ShapeDtypeStruct((NC, N_PAD, HD), jnp.float32),
    mesh=_MESH,
    compiler_params=pltpu.CompilerParams(use_tc_tiling_on_sc=False),
    scratch_types=[
        pltpu.VMEM((2, NCHUNK, CH), jnp.int32),
        pltpu.VMEM((2, CH, HD), jnp.float32),
        pltpu.VMEM_SHARED((N_PAD, HD), jnp.float32),
        pltpu.SemaphoreType.DMA,
        pltpu.SemaphoreType.DMA,
        pltpu.SemaphoreType.DMA,
        pltpu.SemaphoreType.DMA,
    ],
)
def _scatter_kernel(g_hbm, idx_hbm, zeros_hbm, out_hbm,
                    idx_all, bufs, acc_sh,
                    gsem0, gsem1, ssem0, ssem1):
    c = lax.axis_index("c")
    s = lax.axis_index("s")
    buf0 = bufs.at[0]
    buf1 = bufs.at[1]
    g_half = g_hbm.at[c]
    pltpu.sync_copy(idx_hbm.at[s], idx_all)
    pltpu.sync_copy(zeros_hbm, acc_sh.at[pl.ds(s * RPT, RPT)])
    plsc.subcore_barrier()

    def gather(i, buf, sem):
        pltpu.async_copy(g_half.at[idx_all.at[0, i]], buf, sem)

    def gather_wait(buf, sem):
        pltpu.make_async_copy(g_half.at[idx_all.at[0, 0]], buf, sem).wait()

    def scatter(i, buf, sem):
        pltpu.async_copy(buf, acc_sh.at[idx_all.at[1, i]], sem, add=True)

    def scatter_wait(buf, sem):
        pltpu.make_async_copy(buf, acc_sh.at[idx_all.at[1, 0]], sem).wait()

    gather(0, buf0, gsem0)
    gather(1, buf1, gsem1)
    gather_wait(buf0, gsem0)
    scatter(0, buf0, ssem0)

    def body(j, carry):
        i0 = 2 * j
        scatter_wait(buf0, ssem0)
        gather(i0, buf0, gsem0)
        gather_wait(buf1, gsem1)
        scatter(i0 - 1, buf1, ssem1)
        scatter_wait(buf1, ssem1)
        gather(jnp.minimum(i0 + 1, NCHUNK - 1), buf1, gsem1)
        gather_wait(buf0, gsem0)
        scatter(i0, buf0, ssem0)
        return carry

    lax.fori_loop(1, (NCHUNK + 1) // 2, body, 0)
    if NCHUNK % 2 == 0:
        gather_wait(buf1, gsem1)
        scatter(NCHUNK - 1, buf1, ssem1)
        scatter_wait(buf0, ssem0)
        scatter_wait(buf1, ssem1)
    else:
        gather_wait(buf1, gsem1)
        scatter_wait(buf0, ssem0)
    plsc.subcore_barrier()
    pltpu.sync_copy(acc_sh.at[pl.ds(s * RPT, RPT)],
                    out_hbm.at[c, pl.ds(s * RPT, RPT)])


def _k1_body(x_ref, w_ref, degp_ref, o_ref):
    deg = degp_ref[0] + degp_ref[1] + 1.0
    dis = lax.rsqrt(deg)
    h = jnp.dot(x_ref[...], w_ref[...], preferred_element_type=jnp.float32)
    g = h * dis
    o_ref[0] = g[:, :HD]
    o_ref[1] = g[:, HD:]


def _k3_body(acc_ref, g_ref, degp_ref, b_ref, w_ref, o_ref):
    deg = degp_ref[0] + degp_ref[1] + 1.0
    dis = lax.rsqrt(deg)
    a = jnp.concatenate([acc_ref[0], acc_ref[1]], axis=1)
    g = jnp.concatenate([g_ref[0], g_ref[1]], axis=1)
    t = dis * (a + g) + b_ref[...]
    t = jnp.maximum(t, 0.0)
    h = jnp.dot(t, w_ref[...], preferred_element_type=jnp.float32)
    g2 = h * dis
    o_ref[0] = g2[:, :HD]
    o_ref[1] = g2[:, HD:]


def _k5_body(acc_ref, g_ref, degp_ref, b_ref, o_ref):
    deg = degp_ref[0] + degp_ref[1] + 1.0
    dis = lax.rsqrt(deg)
    a = jnp.concatenate([acc_ref[0], acc_ref[1]], axis=1)
    g = jnp.concatenate([g_ref[0], g_ref[1]], axis=1)
    t = dis * (a + g) + b_ref[...]
    o_ref[...] = jnp.maximum(t, 0.0)


_ROWS = pl.BlockSpec((BN, D), lambda i: (i, 0))
_HALVES = pl.BlockSpec((NC, BN, HD), lambda i: (0, i, 0))
_DEGP = pl.BlockSpec((NC, BN, 1), lambda i: (0, i, 0))
_MAT = pl.BlockSpec((D, D), lambda i: (0, 0))
_BIAS = pl.BlockSpec((1, D), lambda i: (0, 0))

_k1 = pl.pallas_call(
    _k1_body, grid=(GRID,),
    in_specs=[_ROWS, _MAT, _DEGP],
    out_specs=_HALVES,
    out_shape=jax.ShapeDtypeStruct((NC, N_PAD, HD), jnp.float32),
)
_k3 = pl.pallas_call(
    _k3_body, grid=(GRID,),
    in_specs=[_HALVES, _HALVES, _DEGP, _BIAS, _MAT],
    out_specs=_HALVES,
    out_shape=jax.ShapeDtypeStruct((NC, N_PAD, HD), jnp.float32),
)
_k5 = pl.pallas_call(
    _k5_body, grid=(GRID,),
    in_specs=[_HALVES, _HALVES, _DEGP, _BIAS],
    out_specs=_ROWS,
    out_shape=jax.ShapeDtypeStruct((N_PAD, D), jnp.float32),
)


def kernel(x, e, W1, b1, W2, b2):
    idx = jnp.transpose(e.reshape(2, NS, NCHUNK, CH), (1, 0, 2, 3))
    dst32 = e[1].reshape(NW, NCHUNK_D, CHD)
    x_pad = jnp.pad(x, ((0, N_PAD - N), (0, 0)))
    zeros1 = jnp.zeros((RPT,), jnp.float32)
    zeros2 = jnp.zeros((RPT, HD), jnp.float32)

    degp = _deg_kernel(dst32, zeros1)
    degp = degp.reshape(NC, N_PAD, 1)
    b1r = b1.reshape(1, D)
    b2r = b2.reshape(1, D)

    g1 = _k1(x_pad, W1, degp)
    acc1 = _scatter_kernel(g1, idx, zeros2)
    g2 = _k3(acc1, g1, degp, b1r, W2)
    acc2 = _scatter_kernel(g2, idx, zeros2)
    out = _k5(acc2, g2, degp, b2r)
    return out[:N]


# speedup vs baseline: 44.9720x; 1.7151x over previous
import functools

import jax
import jax.numpy as jnp
from jax import lax
from jax.experimental import pallas as pl
from jax.experimental.pallas import tpu as pltpu
from jax.experimental.pallas import tpu_sc as plsc

N = 10000
E = 320000
D = 128
HD = D // 2

NC = 2
NS = 16
NW = NC * NS
N_PAD = 10240
RPT = N_PAD // NS

EPS = E // NS
CH = 125
NCHUNK = EPS // CH

NCHUNK_D = NCHUNK // NC
NB = 8
NROUND = NCHUNK // NB

BN = 2560
GRID = N_PAD // BN

_MESH = plsc.VectorSubcoreMesh(core_axis_name="c", subcore_axis_name="s")


@functools.partial(
    pl.kernel,
    out_type=jax.ShapeDtypeStruct((NC, N_PAD), jnp.float32),
    mesh=_MESH,
    compiler_params=pltpu.CompilerParams(use_tc_tiling_on_sc=False),
    scratch_types=[
        pltpu.VMEM((NCHUNK_D, CH), jnp.int32),
        pltpu.VMEM((CH,), jnp.float32),
        pltpu.VMEM_SHARED((N_PAD,), jnp.float32),
        pltpu.SemaphoreType.DMA,
    ],
)
def _deg_kernel(idx_hbm, zeros_hbm, out_hbm, dst_all, ones_v, deg_sh, sem):
    c = lax.axis_index("c")
    s = lax.axis_index("s")
    for i in range(0, CH - 15, 16):
        ones_v[pl.ds(i, 16)] = jnp.ones((16,), jnp.float32)
    ones_v[pl.ds(CH - 16, 16)] = jnp.ones((16,), jnp.float32)
    pltpu.sync_copy(idx_hbm.at[1, s, pl.ds(c * NCHUNK_D, NCHUNK_D)], dst_all)
    pltpu.sync_copy(zeros_hbm, deg_sh.at[pl.ds(s * RPT, RPT)])
    plsc.subcore_barrier()

    def group(j, carry):
        for b in range(NB):
            pltpu.async_copy(ones_v, deg_sh.at[dst_all.at[NB * j + b]],
                             sem, add=True)
        for b in range(NB):
            pltpu.make_async_copy(ones_v, deg_sh.at[dst_all.at[0]],
                                  sem).wait()
        return carry

    lax.fori_loop(0, NCHUNK_D // NB, group, 0)
    plsc.subcore_barrier()
    pltpu.sync_copy(deg_sh.at[pl.ds(s * RPT, RPT)],
                    out_hbm.at[c, pl.ds(s * RPT, RPT)])


@functools.partial(
    pl.kernel,
    out_type=jax.ShapeDtypeStruct((NC, N_PAD, HD), jnp.bfloat16),
    mesh=_MESH,
    compiler_params=pltpu.CompilerParams(use_tc_tiling_on_sc=False),
    scratch_types=[
        pltpu.VMEM((2, NCHUNK, CH), jnp.int32),
        pltpu.VMEM((NB, CH, HD), jnp.bfloat16),
        pltpu.VMEM_SHARED((N_PAD, HD), jnp.bfloat16),
        [pltpu.SemaphoreType.DMA] * NB,
        [pltpu.SemaphoreType.DMA] * NB,
    ],
)
def _scatter_kernel(g_hbm, idx_hbm, zeros_hbm, out_hbm,
                    idx_all, bufs, acc_sh, gsems, ssems):
    c = lax.axis_index("c")
    s = lax.axis_index("s")
    g_half = g_hbm.at[c]
    pltpu.sync_copy(idx_hbm.at[0, s], idx_all.at[0])
    pltpu.sync_copy(idx_hbm.at[1, s], idx_all.at[1])
    pltpu.sync_copy(zeros_hbm, acc_sh.at[pl.ds(s * RPT, RPT)])
    plsc.subcore_barrier()

    def gather(i, b):
        pltpu.async_copy(g_half.at[idx_all.at[0, i]], bufs.at[b], gsems[b])

    def gather_wait(b):
        pltpu.make_async_copy(g_half.at[idx_all.at[0, 0]], bufs.at[b],
                              gsems[b]).wait()

    def scatter(i, b):
        pltpu.async_copy(bufs.at[b], acc_sh.at[idx_all.at[1, i]], ssems[b],
                         add=True)

    def scatter_wait(b):
        pltpu.make_async_copy(bufs.at[b], acc_sh.at[idx_all.at[1, 0]],
                              ssems[b]).wait()

    for b in range(NB):
        gather(b, b)

    def body(j, carry):
        i0 = NB * j
        for b in range(NB):
            gather_wait(b)
            scatter(i0 + b, b)
        for b in range(NB):
            scatter_wait(b)
            gather(jnp.minimum(i0 + NB + b, NCHUNK - 1), b)
        return carry

    lax.fori_loop(0, NROUND, body, 0)
    for b in range(NB):
        gather_wait(b)
    plsc.subcore_barrier()
    pltpu.sync_copy(acc_sh.at[pl.ds(s * RPT, RPT)],
                    out_hbm.at[c, pl.ds(s * RPT, RPT)])


BNR = BN // 128


def _dis3(degp_ref):
    r = pl.program_id(0) * BNR
    deg = (degp_ref[0, pl.ds(r, BNR), :]
           + degp_ref[1, pl.ds(r, BNR), :] + 1.0)
    return lax.rsqrt(deg)[:, :, None]


def _row_scale(m, dis3):
    w = m.shape[1]
    return (m.reshape(BNR, 128, w) * dis3).reshape(BN, w)


def _k1_body(x_ref, w_ref, degp_ref, o_ref):
    dis = _dis3(degp_ref)
    h = jnp.dot(x_ref[...], w_ref[...], preferred_element_type=jnp.float32)
    g = _row_scale(h, dis).astype(jnp.bfloat16)
    o_ref[0] = g[:, :HD]
    o_ref[1] = g[:, HD:]


def _merge_halves(acc_ref, g_ref, b_ref, dis):
    th = []
    for hh in range(NC):
        sh = acc_ref[hh].astype(jnp.float32) + g_ref[hh].astype(jnp.float32)
        sh = _row_scale(sh, dis) + b_ref[0, hh * HD:(hh + 1) * HD][None, :]
        th.append(jnp.maximum(sh, 0.0))
    return th


def _k3_body(acc_ref, g_ref, degp_ref, b_ref, w_ref, o_ref):
    dis = _dis3(degp_ref)
    t0, t1 = _merge_halves(acc_ref, g_ref, b_ref, dis)
    h = (jnp.dot(t0, w_ref[:HD, :], preferred_element_type=jnp.float32)
         + jnp.dot(t1, w_ref[HD:, :], preferred_element_type=jnp.float32))
    g2 = _row_scale(h, dis).astype(jnp.bfloat16)
    o_ref[0] = g2[:, :HD]
    o_ref[1] = g2[:, HD:]


def _k5_body(acc_ref, g_ref, degp_ref, b_ref, o_ref):
    dis = _dis3(degp_ref)
    t0, t1 = _merge_halves(acc_ref, g_ref, b_ref, dis)
    o_ref[:, :HD] = t0
    o_ref[:, HD:] = t1


_ROWS = pl.BlockSpec((BN, D), lambda i: (i, 0))
_HALVES = pl.BlockSpec((NC, BN, HD), lambda i: (0, i, 0))
_DEGP = pl.BlockSpec((NC, N_PAD // 128, 128), lambda i: (0, 0, 0))
_MAT = pl.BlockSpec((D, D), lambda i: (0, 0))
_BIAS = pl.BlockSpec((1, D), lambda i: (0, 0))

_k1 = pl.pallas_call(
    _k1_body, grid=(GRID,),
    in_specs=[_ROWS, _MAT, _DEGP],
    out_specs=_HALVES,
    out_shape=jax.ShapeDtypeStruct((NC, N_PAD, HD), jnp.bfloat16),
)
_k3 = pl.pallas_call(
    _k3_body, grid=(GRID,),
    in_specs=[_HALVES, _HALVES, _DEGP, _BIAS, _MAT],
    out_specs=_HALVES,
    out_shape=jax.ShapeDtypeStruct((NC, N_PAD, HD), jnp.bfloat16),
)
_k5 = pl.pallas_call(
    _k5_body, grid=(GRID,),
    in_specs=[_HALVES, _HALVES, _DEGP, _BIAS],
    out_specs=_ROWS,
    out_shape=jax.ShapeDtypeStruct((N, D), jnp.float32),
)


def kernel(x, e, W1, b1, W2, b2):
    idx = e.reshape(2, NS, NCHUNK, CH)
    zeros1 = jnp.zeros((RPT,), jnp.float32)
    zeros2 = jnp.zeros((RPT, HD), jnp.bfloat16)

    degp = _deg_kernel(idx, zeros1)
    degp = degp.reshape(NC, N_PAD // 128, 128)
    b1r = b1.reshape(1, D)
    b2r = b2.reshape(1, D)

    g1 = _k1(x, W1, degp)
    acc1 = _scatter_kernel(g1, idx, zeros2)
    g2 = _k3(acc1, g1, degp, b1r, W2)
    acc2 = _scatter_kernel(g2, idx, zeros2)
    return _k5(acc2, g2, degp, b2r)
